# baseline (speedup 1.0000x reference)
import functools
import math

import jax
import jax.numpy as jnp
import numpy as np
from jax import lax
from jax.experimental import pallas as pl
from jax.experimental.pallas import tpu as pltpu
from jax.experimental.pallas import tpu_sc as plsc

_BATCH = 4096
_SEQ = 200
_D = 64
_B = _BATCH * _SEQ

_NC = 2
_NS = 16
_NW = _NC * _NS
_PLANES = _SEQ * _D
_PPW = _PLANES // _NW
_CP = 8
_NCHUNK = _PPW // _CP
_NBUF = 3
_TCOLS = 128


def _idx_body(ts_ref, idx_ref):
    hours = ts_ref[...] // 3600
    cur = hours[_SEQ - 1:_SEQ, :]
    n = ((cur - hours) + 1).astype(jnp.float32)
    d = jnp.log(n) / np.float32(math.log(2))
    idx_ref[...] = jnp.ceil(d).astype(jnp.int32)


def _compute_idx(ts_t):
    blk = 512
    return pl.pallas_call(
        _idx_body,
        out_shape=jax.ShapeDtypeStruct((_SEQ, _BATCH), jnp.int32),
        grid=(_BATCH // blk,),
        in_specs=[pl.BlockSpec((_SEQ, blk), lambda i: (0, i))],
        out_specs=pl.BlockSpec((_SEQ, blk), lambda i: (0, i)),
    )(ts_t)


def _gather_body(idx_hbm, table_hbm, out_hbm, idx_v, table_v, rows_v, ssem):
    wid = lax.axis_index("s") * _NC + lax.axis_index("c")
    p0w = wid * _PPW
    pltpu.sync_copy(table_hbm.at[:, pl.ds(0, _TCOLS)], table_v)

    def chunk(c, carry):
        buf = lax.rem(c, _NBUF)
        p0 = p0w + c * _CP
        s = lax.shift_right_logical(p0, 6)
        d0 = pl.multiple_of(lax.rem(p0, 64), _CP)

        @pl.when(jnp.logical_or(c == 0, d0 == 0))
        def _():
            pltpu.sync_copy(idx_hbm.at[s], idx_v)

        def block(blk, c2):
            vi = idx_v[0, pl.ds(blk * 16, 16)]
            for p in range(_CP):
                v = plsc.load_gather(
                    table_v, [jnp.full((16,), d0 + p, jnp.int32), vi]
                )
                rows_v[buf, p, pl.ds(blk * 16, 16)] = v
            return c2

        lax.fori_loop(0, _BATCH // 16, block, 0)

        @pl.when(c >= _NBUF)
        def _():
            pltpu.make_async_copy(
                rows_v.at[buf], out_hbm.at[0, pl.ds(0, _CP)], ssem
            ).wait()

        pltpu.async_copy(
            rows_v.at[buf], out_hbm.at[s, pl.ds(d0, _CP)], ssem
        )
        return carry

    lax.fori_loop(0, _NCHUNK, chunk, 0)
    for _ in range(_NBUF):
        pltpu.make_async_copy(
            rows_v.at[0], out_hbm.at[0, pl.ds(0, _CP)], ssem
        ).wait()


def _gather_sc(idx_t3, table_t):
    mesh = plsc.VectorSubcoreMesh(
        core_axis_name="c", subcore_axis_name="s",
        num_cores=_NC, num_subcores=_NS,
    )
    f = functools.partial(
        pl.kernel,
        out_type=jax.ShapeDtypeStruct((_SEQ, _D, _BATCH), jnp.float32),
        mesh=mesh,
        scratch_types=[
            pltpu.VMEM((1, _BATCH), jnp.int32),
            pltpu.VMEM((_D, _TCOLS), jnp.float32),
            pltpu.VMEM((_NBUF, _CP, _BATCH), jnp.float32),
            pltpu.SemaphoreType.DMA,
        ],
        compiler_params=pltpu.CompilerParams(
            use_tc_tiling_on_sc=True, needs_layout_passes=False
        ),
    )(_gather_body)
    return f(idx_t3, table_t)


def kernel(timestamps, te_weight):
    ts_t = timestamps.astype(jnp.int32).T
    idx_t = _compute_idx(ts_t)
    out_t = _gather_sc(idx_t.reshape(_SEQ, 1, _BATCH), te_weight.T)
    return out_t.transpose(2, 0, 1)


# speedup vs baseline: 2.5493x; 2.5493x over previous
import functools
import math

import jax
import jax.numpy as jnp
import numpy as np
from jax import lax
from jax.experimental import pallas as pl
from jax.experimental.pallas import tpu as pltpu
from jax.experimental.pallas import tpu_sc as plsc

_BATCH = 4096
_SEQ = 200
_D = 64
_B = _BATCH * _SEQ

_NC = 2
_NS = 16
_NW = _NC * _NS
_PLANES = _SEQ * _D
_PPW = _PLANES // _NW
_CP = 8
_NCHUNK = _PPW // _CP
_NBUF = 3
_TCOLS = 128


def _idx_body(ts_ref, idx_ref):
    hours = ts_ref[...] // 3600
    cur = hours[_SEQ - 1:_SEQ, :]
    n = ((cur - hours) + 1).astype(jnp.float32)
    d = jnp.log(n) / np.float32(math.log(2))
    idx_ref[...] = jnp.ceil(d).astype(jnp.int32)


def _compute_idx(ts_t):
    blk = 512
    return pl.pallas_call(
        _idx_body,
        out_shape=jax.ShapeDtypeStruct((_SEQ, _BATCH), jnp.int32),
        grid=(_BATCH // blk,),
        in_specs=[pl.BlockSpec((_SEQ, blk), lambda i: (0, i))],
        out_specs=pl.BlockSpec((_SEQ, blk), lambda i: (0, i)),
    )(ts_t)


def _gather_body(idx_hbm, table_hbm, out_hbm, idx_v, table_v, rows_v, ssem):
    wid = lax.axis_index("s") * _NC + lax.axis_index("c")
    p0w = wid * _PPW
    pltpu.sync_copy(table_hbm.at[:, pl.ds(0, _TCOLS)], table_v)

    def chunk(c, carry):
        buf = lax.rem(c, _NBUF)
        p0 = p0w + c * _CP
        s = lax.shift_right_logical(p0, 6)
        d0 = pl.multiple_of(lax.rem(p0, 64), _CP)

        @pl.when(jnp.logical_or(c == 0, d0 == 0))
        def _():
            pltpu.sync_copy(idx_hbm.at[s], idx_v)

        trows = [table_v.at[d0 + p] for p in range(_CP)]

        def block(blk, c2):
            vi = idx_v[0, pl.ds(blk * 16, 16)]
            vals = [plsc.load_gather(trows[p], [vi]) for p in range(_CP)]
            for p in range(_CP):
                rows_v[buf, p, pl.ds(blk * 16, 16)] = vals[p]
            return c2

        lax.fori_loop(0, _BATCH // 16, block, 0)

        @pl.when(c >= _NBUF)
        def _():
            pltpu.make_async_copy(
                rows_v.at[buf], out_hbm.at[0, pl.ds(0, _CP)], ssem
            ).wait()

        pltpu.async_copy(
            rows_v.at[buf], out_hbm.at[s, pl.ds(d0, _CP)], ssem
        )
        return carry

    lax.fori_loop(0, _NCHUNK, chunk, 0)
    for _ in range(_NBUF):
        pltpu.make_async_copy(
            rows_v.at[0], out_hbm.at[0, pl.ds(0, _CP)], ssem
        ).wait()


def _gather_sc(idx_t3, table_t):
    mesh = plsc.VectorSubcoreMesh(
        core_axis_name="c", subcore_axis_name="s",
        num_cores=_NC, num_subcores=_NS,
    )
    f = functools.partial(
        pl.kernel,
        out_type=jax.ShapeDtypeStruct((_SEQ, _D, _BATCH), jnp.float32),
        mesh=mesh,
        scratch_types=[
            pltpu.VMEM((1, _BATCH), jnp.int32),
            pltpu.VMEM((_D, _TCOLS), jnp.float32),
            pltpu.VMEM((_NBUF, _CP, _BATCH), jnp.float32),
            pltpu.SemaphoreType.DMA,
        ],
        compiler_params=pltpu.CompilerParams(
            use_tc_tiling_on_sc=True, needs_layout_passes=False
        ),
    )(_gather_body)
    return f(idx_t3, table_t)


def kernel(timestamps, te_weight):
    ts_t = timestamps.astype(jnp.int32).T
    idx_t = _compute_idx(ts_t)
    out_t = _gather_sc(idx_t.reshape(_SEQ, 1, _BATCH), te_weight.T)
    return out_t.transpose(2, 0, 1)
